# Initial kernel scaffold; baseline (speedup 1.0000x reference)
import jax
import jax.numpy as jnp
from jax.experimental import pallas as pl


def kernel(x, edge_index, edge_type, edge_attr, Wl1, Wr1, att1, b1, Wl2, Wr2, att2, b2, lin_W, lin_b):
    raise NotImplementedError("write your pallas kernel here")


# speedup vs baseline: 5.5323x; 5.5323x over previous
import functools

import jax
import jax.numpy as jnp
from jax import lax
from jax.experimental import pallas as pl
from jax.experimental.pallas import tpu as pltpu
from jax.experimental.pallas import tpu_sc as plsc

N = 10000
E = 160000
EP = E + N
DIM_IN = 128
D1 = 512
D2 = 256
HEADS = 8
DIM_H = 64

NPAD = 10240
EPAD = EP + 32
G = 8
RN1 = 160
RN2 = 320
NWORK = 32


def _tc_pre_body(x_ref, wl_ref, wr_ref, abd_ref, xl_ref, xr_ref, stab_ref):
    x = x_ref[...]
    xl = jnp.dot(x, wl_ref[...], preferred_element_type=jnp.float32)
    xr = jnp.dot(x, wr_ref[...], preferred_element_type=jnp.float32)
    m = xl + xr
    z = jnp.maximum(m, 0.2 * m)
    stab_ref[...] = jnp.dot(z, abd_ref[...], preferred_element_type=jnp.float32)
    xl_ref[...] = xl
    xr_ref[...] = xr


def _tc_pre(xp, W_l, W_r, att_bd, din, dout):
    blk = 256
    return pl.pallas_call(
        _tc_pre_body,
        grid=(NPAD // blk,),
        in_specs=[
            pl.BlockSpec((blk, din), lambda i: (i, 0)),
            pl.BlockSpec((din, dout), lambda i: (0, 0)),
            pl.BlockSpec((din, dout), lambda i: (0, 0)),
            pl.BlockSpec((dout, 16), lambda i: (0, 0)),
        ],
        out_specs=[
            pl.BlockSpec((blk, dout), lambda i: (i, 0)),
            pl.BlockSpec((blk, dout), lambda i: (i, 0)),
            pl.BlockSpec((blk, 16), lambda i: (i, 0)),
        ],
        out_shape=[
            jax.ShapeDtypeStruct((NPAD, dout), jnp.float32),
            jax.ShapeDtypeStruct((NPAD, dout), jnp.float32),
            jax.ShapeDtypeStruct((NPAD, 16), jnp.float32),
        ],
    )(xp, W_l, W_r, att_bd)


def _tc_mid_body(num_ref, den_ref, b1_ref, wl_ref, wr_ref, abd_ref, exp_ref,
                 xl_ref, xr_ref, stab_ref):
    dexp = jnp.dot(den_ref[...], exp_ref[...], preferred_element_type=jnp.float32)
    h = num_ref[...] / (dexp + 1e-16) + b1_ref[...]
    h = jnp.maximum(h, 0.01 * h)
    xl = jnp.dot(h, wl_ref[...], preferred_element_type=jnp.float32)
    xr = jnp.dot(h, wr_ref[...], preferred_element_type=jnp.float32)
    m = xl + xr
    z = jnp.maximum(m, 0.2 * m)
    stab_ref[...] = jnp.dot(z, abd_ref[...], preferred_element_type=jnp.float32)
    xl_ref[...] = xl
    xr_ref[...] = xr


def _tc_mid(num1, den1, b1r, Wl2, Wr2, att_bd2, expand):
    blk = 256
    return pl.pallas_call(
        _tc_mid_body,
        grid=(NPAD // blk,),
        in_specs=[
            pl.BlockSpec((blk, D1), lambda i: (i, 0)),
            pl.BlockSpec((blk, 16), lambda i: (i, 0)),
            pl.BlockSpec((1, D1), lambda i: (0, 0)),
            pl.BlockSpec((D1, D2), lambda i: (0, 0)),
            pl.BlockSpec((D1, D2), lambda i: (0, 0)),
            pl.BlockSpec((D2, 16), lambda i: (0, 0)),
            pl.BlockSpec((16, D1), lambda i: (0, 0)),
        ],
        out_specs=[
            pl.BlockSpec((blk, D2), lambda i: (i, 0)),
            pl.BlockSpec((blk, D2), lambda i: (i, 0)),
            pl.BlockSpec((blk, 16), lambda i: (i, 0)),
        ],
        out_shape=[
            jax.ShapeDtypeStruct((NPAD, D2), jnp.float32),
            jax.ShapeDtypeStruct((NPAD, D2), jnp.float32),
            jax.ShapeDtypeStruct((NPAD, 16), jnp.float32),
        ],
    )(num1, den1, b1r, Wl2, Wr2, att_bd2, expand)


def _tc_fin_body(num_ref, den_ref, b2_ref, lw_ref, lb_ref, out_ref, acc_ref):
    i = pl.program_id(0)
    d = den_ref[:, 0:1]
    h2 = num_ref[...] / (d + 1e-16)
    row = i * 256 + lax.broadcasted_iota(jnp.int32, (256, 1), 0)
    h2 = jnp.where(row < N, h2, 0.0)
    psum = jnp.sum(h2, axis=0, keepdims=True)

    @pl.when(i == 0)
    def _():
        acc_ref[...] = jnp.zeros_like(acc_ref)

    acc_ref[...] += psum

    @pl.when(i == NPAD // 256 - 1)
    def _():
        g = acc_ref[...] + float(N) * b2_ref[...]
        out_ref[...] = (jnp.dot(g, lw_ref[...], preferred_element_type=jnp.float32)
                        + lb_ref[...])


def _tc_fin(num2, den2, b2r, lin_W, lin_b2):
    return pl.pallas_call(
        _tc_fin_body,
        grid=(NPAD // 256,),
        in_specs=[
            pl.BlockSpec((256, D2), lambda i: (i, 0)),
            pl.BlockSpec((256, 16), lambda i: (i, 0)),
            pl.BlockSpec((1, D2), lambda i: (0, 0)),
            pl.BlockSpec((D2, 2), lambda i: (0, 0)),
            pl.BlockSpec((1, 2), lambda i: (0, 0)),
        ],
        out_specs=pl.BlockSpec((1, 2), lambda i: (0, 0)),
        out_shape=jax.ShapeDtypeStruct((1, 2), jnp.float32),
        scratch_shapes=[pltpu.VMEM((1, D2), jnp.float32)],
    )(num2, den2, b2r, lin_W, lin_b2)


def _vextract(ref, idx):
    return ref[pl.ds(idx, 16)][0]


def _sc_edge1_body(xl_hbm, xr_hbm, stab_hbm, srcs_hbm, dsts_hbm, roff_hbm,
                   att_hbm, num_hbm, den_hbm,
                   acc_v, den_v, stab_v, xlr_v, xrr_v, isrc_v,
                   idstx_v, roff_v, att_v, sem1, sem2):
    wid = lax.axis_index("s") * 2 + lax.axis_index("c")
    pltpu.sync_copy(roff_hbm, roff_v)
    pltpu.sync_copy(att_hbm, att_v)
    attregs = [att_v[pl.ds(16 * k, 16)] for k in range(32)]
    lane = lax.iota(jnp.int32, 16)

    for rr in range(2):
        r = wid * 2 + rr
        node_base = r * RN1
        o_lo = _vextract(roff_v, r)
        o_hi = _vextract(roff_v, r + 1)
        a_lo = (o_lo // 8) * 8
        nch = (o_hi - a_lo + (G - 1)) // G

        def _zero(i, _):
            for k in range(D1 // 16):
                acc_v[i, pl.ds(16 * k, 16)] = jnp.zeros((16,), jnp.float32)
            den_v[pl.ds(16 * i, 16)] = jnp.zeros((16,), jnp.float32)
            return 0
        lax.fori_loop(0, RN1, _zero, 0)
        pltpu.sync_copy(stab_hbm.at[pl.ds(node_base * 16, RN1 * 16)], stab_v)

        def _chunk(c, _):
            base = a_lo + c * G
            pltpu.sync_copy(srcs_hbm.at[pl.ds(base, G)], isrc_v)
            pltpu.sync_copy(dsts_hbm.at[pl.ds(base, G)],
                            idstx_v.at[pl.ds(0, G)])
            cp1 = pltpu.async_copy(xl_hbm.at[isrc_v], xlr_v, sem1)
            cp2 = pltpu.async_copy(xr_hbm.at[idstx_v.at[pl.ds(0, G)]], xrr_v, sem2)
            cp1.wait()
            cp2.wait()

            def _edge(e, _):
                dstid = _vextract(idstx_v, e)
                ge = base + e
                valid = (ge >= o_lo) & (ge < o_hi)
                scale = jnp.where(valid, 1.0, 0.0)
                dloc = jnp.clip(dstid - node_base, 0, RN1 - 1)
                s = [jnp.zeros((16,), jnp.float32) for _ in range(HEADS)]
                for k in range(32):
                    xlk = xlr_v[e, pl.ds(16 * k, 16)]
                    xrk = xrr_v[e, pl.ds(16 * k, 16)]
                    m = xlk + xrk
                    z = jnp.maximum(m, 0.2 * m)
                    s[k // 4] = s[k // 4] + z * attregs[k]
                lvec = jnp.zeros((16,), jnp.float32)
                for h in range(HEADS):
                    lh = jnp.broadcast_to(jnp.sum(s[h]), (16,))
                    lvec = lvec + jnp.where(lane == h, lh, 0.0)
                pv = jnp.exp(lvec - stab_v[pl.ds(dloc * 16, 16)]) * scale
                plsc.addupdate(den_v.at[pl.ds(dloc * 16, 16)], pv)
                for h in range(HEADS):
                    ph = jnp.broadcast_to(pv[h], (16,))
                    for kk in range(4):
                        k = h * 4 + kk
                        xlk = xlr_v[e, pl.ds(16 * k, 16)]
                        plsc.addupdate(acc_v.at[dloc, pl.ds(16 * k, 16)], xlk * ph)
                return 0

            lax.fori_loop(0, G, _edge, 0)
            return 0

        lax.fori_loop(0, nch, _chunk, 0)
        pltpu.sync_copy(acc_v, num_hbm.at[pl.ds(node_base, RN1)])
        pltpu.sync_copy(den_v, den_hbm.at[pl.ds(node_base * 16, RN1 * 16)])


def _sc_edge1(xl, xr, stab, srcs, dsts, roff, attv):
    mesh = plsc.VectorSubcoreMesh(core_axis_name="c", subcore_axis_name="s")
    f = pl.kernel(
        _sc_edge1_body,
        out_type=[
            jax.ShapeDtypeStruct((NPAD, D1), jnp.float32),
            jax.ShapeDtypeStruct((NPAD * 16,), jnp.float32),
        ],
        mesh=mesh,
        compiler_params=pltpu.CompilerParams(needs_layout_passes=False),
        scratch_types=[
            pltpu.VMEM((RN1, D1), jnp.float32),
            pltpu.VMEM((RN1 * 16,), jnp.float32),
            pltpu.VMEM((RN1 * 16,), jnp.float32),
            pltpu.VMEM((G, D1), jnp.float32),
            pltpu.VMEM((G, D1), jnp.float32),
            pltpu.VMEM((G,), jnp.int32),
            pltpu.VMEM((G + 16,), jnp.int32),
            pltpu.VMEM((96,), jnp.int32),
            pltpu.VMEM((512,), jnp.float32),
            pltpu.SemaphoreType.DMA,
            pltpu.SemaphoreType.DMA,
        ],
    )
    return f(xl, xr, stab, srcs, dsts, roff, attv)


def _sc_edge2_body(xl_hbm, xr_hbm, stab_hbm, srcs_hbm, dsts_hbm, roff_hbm,
                   att_hbm, num_hbm, den_hbm,
                   acc_v, den_v, stab_v, xlr_v, xrr_v, isrc_v,
                   idstx_v, roff_v, att_v, sem1, sem2):
    wid = lax.axis_index("s") * 2 + lax.axis_index("c")
    pltpu.sync_copy(roff_hbm, roff_v)
    pltpu.sync_copy(att_hbm, att_v)
    attregs = [att_v[pl.ds(16 * k, 16)] for k in range(16)]

    r = wid
    node_base = r * RN2
    o_lo = _vextract(roff_v, r)
    o_hi = _vextract(roff_v, r + 1)
    a_lo = (o_lo // 8) * 8
    nch = (o_hi - a_lo + (G - 1)) // G

    def _zero(i, _):
        for k in range(D2 // 16):
            acc_v[i, pl.ds(16 * k, 16)] = jnp.zeros((16,), jnp.float32)
        den_v[pl.ds(16 * i, 16)] = jnp.zeros((16,), jnp.float32)
        return 0
    lax.fori_loop(0, RN2, _zero, 0)
    pltpu.sync_copy(stab_hbm.at[pl.ds(node_base * 16, RN2 * 16)], stab_v)

    def _chunk(c, _):
        base = a_lo + c * G
        pltpu.sync_copy(srcs_hbm.at[pl.ds(base, G)], isrc_v)
        pltpu.sync_copy(dsts_hbm.at[pl.ds(base, G)], idstx_v.at[pl.ds(0, G)])
        cp1 = pltpu.async_copy(xl_hbm.at[isrc_v], xlr_v, sem1)
        cp2 = pltpu.async_copy(xr_hbm.at[idstx_v.at[pl.ds(0, G)]], xrr_v, sem2)
        cp1.wait()
        cp2.wait()

        def _edge(e, _):
            dstid = _vextract(idstx_v, e)
            ge = base + e
            valid = (ge >= o_lo) & (ge < o_hi)
            scale = jnp.where(valid, 1.0, 0.0)
            dloc = jnp.clip(dstid - node_base, 0, RN2 - 1)
            s = jnp.zeros((16,), jnp.float32)
            for k in range(16):
                xlk = xlr_v[e, pl.ds(16 * k, 16)]
                xrk = xrr_v[e, pl.ds(16 * k, 16)]
                m = xlk + xrk
                z = jnp.maximum(m, 0.2 * m)
                s = s + z * attregs[k]
            logit = jnp.sum(s) - stab_v[pl.ds(dloc * 16, 16)][0]
            pv = jnp.exp(jnp.broadcast_to(logit, (16,))) * scale
            plsc.addupdate(den_v.at[pl.ds(dloc * 16, 16)], pv)
            for k in range(16):
                xlk = xlr_v[e, pl.ds(16 * k, 16)]
                plsc.addupdate(acc_v.at[dloc, pl.ds(16 * k, 16)], xlk * pv)
            return 0

        lax.fori_loop(0, G, _edge, 0)
        return 0

    lax.fori_loop(0, nch, _chunk, 0)
    pltpu.sync_copy(acc_v, num_hbm.at[pl.ds(node_base, RN2)])
    pltpu.sync_copy(den_v, den_hbm.at[pl.ds(node_base * 16, RN2 * 16)])


def _sc_edge2(xl, xr, stab, srcs, dsts, roff, attv):
    mesh = plsc.VectorSubcoreMesh(core_axis_name="c", subcore_axis_name="s")
    f = pl.kernel(
        _sc_edge2_body,
        out_type=[
            jax.ShapeDtypeStruct((NPAD, D2), jnp.float32),
            jax.ShapeDtypeStruct((NPAD * 16,), jnp.float32),
        ],
        mesh=mesh,
        compiler_params=pltpu.CompilerParams(needs_layout_passes=False),
        scratch_types=[
            pltpu.VMEM((RN2, D2), jnp.float32),
            pltpu.VMEM((RN2 * 16,), jnp.float32),
            pltpu.VMEM((RN2 * 16,), jnp.float32),
            pltpu.VMEM((G, D2), jnp.float32),
            pltpu.VMEM((G, D2), jnp.float32),
            pltpu.VMEM((G,), jnp.int32),
            pltpu.VMEM((G + 16,), jnp.int32),
            pltpu.VMEM((96,), jnp.int32),
            pltpu.VMEM((256,), jnp.float32),
            pltpu.SemaphoreType.DMA,
            pltpu.SemaphoreType.DMA,
        ],
    )
    return f(xl, xr, stab, srcs, dsts, roff, attv)


def kernel(x, edge_index, edge_type, edge_attr, Wl1, Wr1, att1, b1,
           Wl2, Wr2, att2, b2, lin_W, lin_b):
    f32 = jnp.float32
    loop = jnp.arange(N, dtype=jnp.int32)
    src = jnp.concatenate([edge_index[0].astype(jnp.int32), loop])
    dst = jnp.concatenate([edge_index[1].astype(jnp.int32), loop])
    order = jnp.argsort(dst)
    srcs = src[order]
    dsts = dst[order]
    roff1 = jnp.searchsorted(dsts, jnp.arange(65, dtype=jnp.int32) * RN1,
                             side="left").astype(jnp.int32)
    roff1 = jnp.concatenate([roff1, jnp.zeros((31,), jnp.int32)])
    roff2 = jnp.searchsorted(dsts, jnp.arange(33, dtype=jnp.int32) * RN2,
                             side="left").astype(jnp.int32)
    roff2 = jnp.concatenate([roff2, jnp.zeros((63,), jnp.int32)])
    srcs = jnp.concatenate([srcs, jnp.zeros((EPAD - EP,), jnp.int32)])
    dsts = jnp.concatenate([dsts, jnp.full((EPAD - EP,), NPAD - 1, jnp.int32)])

    xp = jnp.pad(x.astype(f32), ((0, NPAD - N), (0, 0)))

    a1f = att1.reshape(-1).astype(f32)
    c1 = jnp.arange(D1) // DIM_H
    h1 = jnp.arange(16)
    att_bd1 = jnp.where(h1[None, :] == c1[:, None], a1f[:, None], 0.0)
    a2f = att2.reshape(-1).astype(f32)
    att_bd2 = jnp.where(h1[None, :] == 0, a2f[:, None], 0.0)
    expand = (h1[:, None] == c1[None, :]).astype(f32)

    att1v = a1f
    att2v = a2f

    xl1, xr1, stab1 = _tc_pre(xp, Wl1.astype(f32), Wr1.astype(f32), att_bd1,
                              DIM_IN, D1)
    num1, den1 = _sc_edge1(xl1, xr1, stab1.reshape(-1), srcs, dsts, roff1,
                           att1v)
    xl2, xr2, stab2 = _tc_mid(num1, den1.reshape(NPAD, 16),
                              b1.reshape(1, D1).astype(f32),
                              Wl2.astype(f32), Wr2.astype(f32), att_bd2, expand)
    num2, den2 = _sc_edge2(xl2, xr2, stab2.reshape(-1), srcs, dsts, roff2,
                           att2v)
    out = _tc_fin(num2, den2.reshape(NPAD, 16), b2.reshape(1, D2).astype(f32),
                  lin_W.astype(f32), lin_b.reshape(1, 2).astype(f32))
    return out


# speedup vs baseline: 9.8790x; 1.7857x over previous
import functools

import jax
import jax.numpy as jnp
from jax import lax
from jax.experimental import pallas as pl
from jax.experimental.pallas import tpu as pltpu
from jax.experimental.pallas import tpu_sc as plsc

N = 10000
E = 160000
EP = E + N
DIM_IN = 128
D1 = 512
D2 = 256
HEADS = 8
DIM_H = 64

NPAD = 10240
EPAD = EP + 144
G = 16
RN1 = 160
RN2 = 320
NWORK = 32


def _tc_pre_body(x_ref, wl_ref, wr_ref, abd_ref, xl_ref, xr_ref, stab_ref):
    x = x_ref[...]
    xl = jnp.dot(x, wl_ref[...], preferred_element_type=jnp.float32)
    xr = jnp.dot(x, wr_ref[...], preferred_element_type=jnp.float32)
    m = xl + xr
    z = jnp.maximum(m, 0.2 * m)
    stab_ref[...] = jnp.dot(z, abd_ref[...], preferred_element_type=jnp.float32)
    xl_ref[...] = xl
    xr_ref[...] = xr


def _tc_pre(xp, W_l, W_r, att_bd, din, dout):
    blk = 256
    return pl.pallas_call(
        _tc_pre_body,
        grid=(NPAD // blk,),
        in_specs=[
            pl.BlockSpec((blk, din), lambda i: (i, 0)),
            pl.BlockSpec((din, dout), lambda i: (0, 0)),
            pl.BlockSpec((din, dout), lambda i: (0, 0)),
            pl.BlockSpec((dout, 16), lambda i: (0, 0)),
        ],
        out_specs=[
            pl.BlockSpec((blk, dout), lambda i: (i, 0)),
            pl.BlockSpec((blk, dout), lambda i: (i, 0)),
            pl.BlockSpec((blk, 16), lambda i: (i, 0)),
        ],
        out_shape=[
            jax.ShapeDtypeStruct((NPAD, dout), jnp.float32),
            jax.ShapeDtypeStruct((NPAD, dout), jnp.float32),
            jax.ShapeDtypeStruct((NPAD, 16), jnp.float32),
        ],
    )(xp, W_l, W_r, att_bd)


def _tc_mid_body(num_ref, den_ref, b1_ref, wl_ref, wr_ref, abd_ref, exp_ref,
                 xl_ref, xr_ref, stab_ref):
    dexp = jnp.dot(den_ref[...], exp_ref[...], preferred_element_type=jnp.float32)
    h = num_ref[...] / (dexp + 1e-16) + b1_ref[...]
    h = jnp.maximum(h, 0.01 * h)
    xl = jnp.dot(h, wl_ref[...], preferred_element_type=jnp.float32)
    xr = jnp.dot(h, wr_ref[...], preferred_element_type=jnp.float32)
    m = xl + xr
    z = jnp.maximum(m, 0.2 * m)
    stab_ref[...] = jnp.dot(z, abd_ref[...], preferred_element_type=jnp.float32)
    xl_ref[...] = xl
    xr_ref[...] = xr


def _tc_mid(num1, den1, b1r, Wl2, Wr2, att_bd2, expand):
    blk = 256
    return pl.pallas_call(
        _tc_mid_body,
        grid=(NPAD // blk,),
        in_specs=[
            pl.BlockSpec((blk, D1), lambda i: (i, 0)),
            pl.BlockSpec((blk, 16), lambda i: (i, 0)),
            pl.BlockSpec((1, D1), lambda i: (0, 0)),
            pl.BlockSpec((D1, D2), lambda i: (0, 0)),
            pl.BlockSpec((D1, D2), lambda i: (0, 0)),
            pl.BlockSpec((D2, 16), lambda i: (0, 0)),
            pl.BlockSpec((16, D1), lambda i: (0, 0)),
        ],
        out_specs=[
            pl.BlockSpec((blk, D2), lambda i: (i, 0)),
            pl.BlockSpec((blk, D2), lambda i: (i, 0)),
            pl.BlockSpec((blk, 16), lambda i: (i, 0)),
        ],
        out_shape=[
            jax.ShapeDtypeStruct((NPAD, D2), jnp.float32),
            jax.ShapeDtypeStruct((NPAD, D2), jnp.float32),
            jax.ShapeDtypeStruct((NPAD, 16), jnp.float32),
        ],
    )(num1, den1, b1r, Wl2, Wr2, att_bd2, expand)


def _tc_fin_body(num_ref, den_ref, b2_ref, lw_ref, lb_ref, out_ref, acc_ref):
    i = pl.program_id(0)
    d = den_ref[:, 0:1]
    h2 = num_ref[...] / (d + 1e-16)
    row = i * 256 + lax.broadcasted_iota(jnp.int32, (256, 1), 0)
    h2 = jnp.where(row < N, h2, 0.0)
    psum = jnp.sum(h2, axis=0, keepdims=True)

    @pl.when(i == 0)
    def _():
        acc_ref[...] = jnp.zeros_like(acc_ref)

    acc_ref[...] += psum

    @pl.when(i == NPAD // 256 - 1)
    def _():
        g = acc_ref[...] + float(N) * b2_ref[...]
        out_ref[...] = (jnp.dot(g, lw_ref[...], preferred_element_type=jnp.float32)
                        + lb_ref[...])


def _tc_fin(num2, den2, b2r, lin_W, lin_b2):
    return pl.pallas_call(
        _tc_fin_body,
        grid=(NPAD // 256,),
        in_specs=[
            pl.BlockSpec((256, D2), lambda i: (i, 0)),
            pl.BlockSpec((256, 16), lambda i: (i, 0)),
            pl.BlockSpec((1, D2), lambda i: (0, 0)),
            pl.BlockSpec((D2, 2), lambda i: (0, 0)),
            pl.BlockSpec((1, 2), lambda i: (0, 0)),
        ],
        out_specs=pl.BlockSpec((1, 2), lambda i: (0, 0)),
        out_shape=jax.ShapeDtypeStruct((1, 2), jnp.float32),
        scratch_shapes=[pltpu.VMEM((1, D2), jnp.float32)],
    )(num2, den2, b2r, lin_W, lin_b2)


SCH = 8
SGE = SCH * G


def _vextract(ref, idx):
    return ref[pl.ds(idx, 16)][0]


def _make_sc_body(D, RN, RPT, heads):
    NV = D // 16
    KPH = NV // heads

    def body(xl_hbm, xr_hbm, stab_hbm, srcs_hbm, dsts_hbm, roff_hbm,
             att_hbm, num_hbm, den_hbm,
             acc_v, den_v, stab_v, xlr_v, xrr_v, isrc_v,
             idstx_v, roff_v, att_v, sem_a, sem_b, sem_c, sem_d):
        wid = lax.axis_index("s") * 2 + lax.axis_index("c")
        pltpu.sync_copy(roff_hbm, roff_v)
        pltpu.sync_copy(att_hbm, att_v)
        attregs = [att_v[pl.ds(16 * k, 16)] for k in range(NV)]
        lane = lax.iota(jnp.int32, 16)
        xl_sems = [sem_a, sem_b]
        xr_sems = [sem_c, sem_d]

        def _issue(j):
            slot = j % 2
            pltpu.async_copy(xl_hbm.at[isrc_v.at[pl.ds(j * G, G)]],
                             xlr_v.at[slot], xl_sems[slot])
            pltpu.async_copy(xr_hbm.at[idstx_v.at[pl.ds(j * G, G)]],
                             xrr_v.at[slot], xr_sems[slot])

        def _wait(j):
            slot = j % 2
            pltpu.make_async_copy(xl_hbm.at[isrc_v.at[pl.ds(0, G)]],
                                  xlr_v.at[slot], xl_sems[slot]).wait()
            pltpu.make_async_copy(xr_hbm.at[idstx_v.at[pl.ds(0, G)]],
                                  xrr_v.at[slot], xr_sems[slot]).wait()

        for rr in range(RPT):
            r = wid * RPT + rr
            node_base = r * RN
            o_lo = _vextract(roff_v, r)
            o_hi = _vextract(roff_v, r + 1)
            a_lo = (o_lo // 8) * 8
            nsch = (o_hi - a_lo + (SGE - 1)) // SGE

            def _zero(i, _):
                for k in range(NV):
                    acc_v[i, pl.ds(16 * k, 16)] = jnp.zeros((16,), jnp.float32)
                den_v[pl.ds(16 * i, 16)] = jnp.zeros((16,), jnp.float32)
                return 0
            lax.fori_loop(0, RN, _zero, 0)
            pltpu.sync_copy(stab_hbm.at[pl.ds(node_base * 16, RN * 16)], stab_v)

            def _compute(j, sbase):
                slot = j % 2
                base = sbase + j * G

                def _edge(e, _):
                    dstid = _vextract(idstx_v, j * G + e)
                    ge = base + e
                    valid = (ge >= o_lo) & (ge < o_hi)
                    scale = jnp.where(valid, 1.0, 0.0)
                    dloc = jnp.clip(dstid - node_base, 0, RN - 1)
                    stabrow = stab_v[pl.ds(dloc * 16, 16)]
                    s = [jnp.zeros((16,), jnp.float32) for _ in range(heads)]
                    for k in range(NV):
                        xlk = xlr_v[slot, e, pl.ds(16 * k, 16)]
                        xrk = xrr_v[slot, e, pl.ds(16 * k, 16)]
                        m = xlk + xrk
                        z = jnp.maximum(m, 0.2 * m)
                        s[k // KPH] = s[k // KPH] + z * attregs[k]
                    if heads == 1:
                        ls0 = jnp.sum(s[0]) - stabrow[0]
                        lvec = jnp.broadcast_to(ls0, (16,))
                    else:
                        lvec = jnp.zeros((16,), jnp.float32)
                        for h in range(heads):
                            lh = jnp.sum(s[h]) - stabrow[h]
                            lvec = lvec + jnp.where(
                                lane == h, jnp.broadcast_to(lh, (16,)), 0.0)
                    pv = jnp.exp(lvec) * scale
                    plsc.addupdate(den_v.at[pl.ds(dloc * 16, 16)], pv)
                    for h in range(heads):
                        ph = jnp.broadcast_to(pv[h], (16,)) if heads > 1 else pv
                        for kk in range(KPH):
                            k = h * KPH + kk
                            xlk = xlr_v[slot, e, pl.ds(16 * k, 16)]
                            plsc.addupdate(
                                acc_v.at[dloc, pl.ds(16 * k, 16)], xlk * ph)
                    return 0

                lax.fori_loop(0, G, _edge, 0)

            def _sch_loop(sidx, _):
                sbase = a_lo + sidx * SGE
                pltpu.sync_copy(srcs_hbm.at[pl.ds(sbase, SGE)], isrc_v)
                pltpu.sync_copy(dsts_hbm.at[pl.ds(sbase, SGE)],
                                idstx_v.at[pl.ds(0, SGE)])
                for j in range(SCH):
                    _issue(j)
                    if j >= 1:
                        _wait(j - 1)
                        _compute(j - 1, sbase)
                _wait(SCH - 1)
                _compute(SCH - 1, sbase)
                return 0

            lax.fori_loop(0, nsch, _sch_loop, 0)
            pltpu.sync_copy(acc_v, num_hbm.at[pl.ds(node_base, RN)])
            pltpu.sync_copy(den_v, den_hbm.at[pl.ds(node_base * 16, RN * 16)])

    return body


def _sc_edge(xl, xr, stab, srcs, dsts, roff, attv, D, RN, RPT, heads):
    mesh = plsc.VectorSubcoreMesh(core_axis_name="c", subcore_axis_name="s")
    f = pl.kernel(
        _make_sc_body(D, RN, RPT, heads),
        out_type=[
            jax.ShapeDtypeStruct((NPAD, D), jnp.float32),
            jax.ShapeDtypeStruct((NPAD * 16,), jnp.float32),
        ],
        mesh=mesh,
        compiler_params=pltpu.CompilerParams(needs_layout_passes=False),
        scratch_types=[
            pltpu.VMEM((RN, D), jnp.float32),
            pltpu.VMEM((RN * 16,), jnp.float32),
            pltpu.VMEM((RN * 16,), jnp.float32),
            pltpu.VMEM((2, G, D), jnp.float32),
            pltpu.VMEM((2, G, D), jnp.float32),
            pltpu.VMEM((SGE,), jnp.int32),
            pltpu.VMEM((SGE + 16,), jnp.int32),
            pltpu.VMEM((96,), jnp.int32),
            pltpu.VMEM((D,), jnp.float32),
            pltpu.SemaphoreType.DMA,
            pltpu.SemaphoreType.DMA,
            pltpu.SemaphoreType.DMA,
            pltpu.SemaphoreType.DMA,
        ],
    )
    return f(xl, xr, stab, srcs, dsts, roff, attv)


def kernel(x, edge_index, edge_type, edge_attr, Wl1, Wr1, att1, b1,
           Wl2, Wr2, att2, b2, lin_W, lin_b):
    f32 = jnp.float32
    loop = jnp.arange(N, dtype=jnp.int32)
    src = jnp.concatenate([edge_index[0].astype(jnp.int32), loop])
    dst = jnp.concatenate([edge_index[1].astype(jnp.int32), loop])
    order = jnp.argsort(dst)
    srcs = src[order]
    dsts = dst[order]
    roff1 = jnp.searchsorted(dsts, jnp.arange(65, dtype=jnp.int32) * RN1,
                             side="left").astype(jnp.int32)
    roff1 = jnp.concatenate([roff1, jnp.zeros((31,), jnp.int32)])
    roff2 = jnp.searchsorted(dsts, jnp.arange(33, dtype=jnp.int32) * RN2,
                             side="left").astype(jnp.int32)
    roff2 = jnp.concatenate([roff2, jnp.zeros((63,), jnp.int32)])
    srcs = jnp.concatenate([srcs, jnp.zeros((EPAD - EP,), jnp.int32)])
    dsts = jnp.concatenate([dsts, jnp.full((EPAD - EP,), NPAD - 1, jnp.int32)])

    xp = jnp.pad(x.astype(f32), ((0, NPAD - N), (0, 0)))

    a1f = att1.reshape(-1).astype(f32)
    c1 = jnp.arange(D1) // DIM_H
    h1 = jnp.arange(16)
    att_bd1 = jnp.where(h1[None, :] == c1[:, None], a1f[:, None], 0.0)
    a2f = att2.reshape(-1).astype(f32)
    att_bd2 = jnp.where(h1[None, :] == 0, a2f[:, None], 0.0)
    expand = (h1[:, None] == c1[None, :]).astype(f32)

    att1v = a1f
    att2v = a2f

    xl1, xr1, stab1 = _tc_pre(xp, Wl1.astype(f32), Wr1.astype(f32), att_bd1,
                              DIM_IN, D1)
    num1, den1 = _sc_edge(xl1, xr1, stab1.reshape(-1), srcs, dsts, roff1,
                          att1v, D1, RN1, 2, HEADS)
    xl2, xr2, stab2 = _tc_mid(num1, den1.reshape(NPAD, 16),
                              b1.reshape(1, D1).astype(f32),
                              Wl2.astype(f32), Wr2.astype(f32), att_bd2, expand)
    num2, den2 = _sc_edge(xl2, xr2, stab2.reshape(-1), srcs, dsts, roff2,
                          att2v, D2, RN2, 1, 1)
    out = _tc_fin(num2, den2.reshape(NPAD, 16), b2.reshape(1, D2).astype(f32),
                  lin_W.astype(f32), lin_b.reshape(1, 2).astype(f32))
    return out
